# baseline (speedup 1.0000x reference)
import functools

import jax
import jax.numpy as jnp
from jax import lax
from jax.experimental import pallas as pl
from jax.experimental.pallas import tpu as pltpu
from jax.experimental.pallas import tpu_sc as plsc

N = 10000
E = 320000
D = 128
H = 128
O = 128
G = 128

NC, NS, L = 2, 16, 16
NW = NC * NS
K = 80
R = 4
EPWP = 10240
EP = NW * EPWP
NG = EPWP // (R * K)
NP = 10240
EPW0 = 10240
EPW1 = EP // NS - EPW0
NG0 = EPW0 // (R * K)
NG1 = EPW1 // (R * K)
RPT = NP // NS
DW = 16

_f32 = jnp.float32


def _sc_deg_body(dst_hbm, out_hbm, d0, d1, d2, d3, ones_v, zbuf, acc, *sems):
    c = lax.axis_index("c")
    s = lax.axis_index("s")
    w = c * NS + s
    dbuf = (d0, d1, d2, d3)
    sem_i = sems[:R]
    sem_s = sems[R:]

    for i in range(K // L):
        ones_v[pl.ds(i * L, L)] = jnp.ones((L,), _f32)
    for i in range(RPT // L):
        zbuf[pl.ds(i * L, L)] = jnp.zeros((L,), _f32)
    pltpu.sync_copy(zbuf, acc.at[pl.ds(s * RPT, RPT)])
    plsc.subcore_barrier()

    base = w * EPWP

    def load_idx(chunk, r):
        pltpu.async_copy(dst_hbm.at[pl.ds(base + chunk * K, K)], dbuf[r],
                         sem_i[r])

    def wait_idx(r):
        pltpu.make_async_copy(dst_hbm.at[pl.ds(0, K)], dbuf[r],
                              sem_i[r]).wait()

    def scatter(r):
        pltpu.async_copy(ones_v, acc.at[dbuf[r]], sem_s[r], add=True)

    def wait_sc(r):
        pltpu.make_async_copy(ones_v, acc.at[dbuf[r]], sem_s[r]).wait()

    for r in range(R):
        load_idx(r, r)
    for r in range(R):
        wait_idx(r)
        scatter(r)

    def grp(gi, carry):
        for r in range(R):
            wait_sc(r)
            load_idx(R * (gi + 1) + r, r)
        for r in range(R):
            wait_idx(r)
            scatter(r)
        return carry

    lax.fori_loop(0, NG - 1, grp, 0)
    for r in range(R):
        wait_sc(r)
    plsc.subcore_barrier()
    pltpu.sync_copy(acc.at[pl.ds(s * RPT, RPT)],
                    out_hbm.at[c, pl.ds(s * RPT, RPT)])


@functools.cache
def _sc_degree_kernel():
    return pl.kernel(
        _sc_deg_body,
        out_type=jax.ShapeDtypeStruct((NC, NP), _f32),
        mesh=plsc.VectorSubcoreMesh(core_axis_name="c", subcore_axis_name="s",
                                    num_cores=NC, num_subcores=NS),
        scratch_types=[
            pltpu.VMEM((K,), jnp.int32),
            pltpu.VMEM((K,), jnp.int32),
            pltpu.VMEM((K,), jnp.int32),
            pltpu.VMEM((K,), jnp.int32),
            pltpu.VMEM((K,), _f32),
            pltpu.VMEM((RPT,), _f32),
            pltpu.VMEM_SHARED((NP,), _f32),
        ] + [pltpu.SemaphoreType.DMA] * (2 * R),
    )


def _sc_agg_body(g_hbm, src_hbm, dst_hbm, z_hbm, out_hbm,
                 s0, s1, s2, s3, d0, d1, d2, d3,
                 rows0, rows1, rows2, rows3, acc, *sems):
    c = lax.axis_index("c")
    s = lax.axis_index("s")
    w = c * NS + s
    sbuf = (s0, s1, s2, s3)
    dbuf = (d0, d1, d2, d3)
    rows = (rows0, rows1, rows2, rows3)
    sem_i = sems[:R]
    sem_g = sems[R:]

    with jax.named_scope("agg_zero"):
        pltpu.sync_copy(z_hbm.at[pl.ds(s * RPT, RPT)],
                        acc.at[pl.ds(s * RPT, RPT)])
        plsc.subcore_barrier()

    is0 = c == 0
    epw = jnp.where(is0, EPW0, EPW1)
    ngw = jnp.where(is0, NG0, NG1)
    base = c * (NS * EPW0) + s * epw

    def load_idx(chunk, r):
        off = base + chunk * K
        pltpu.async_copy(src_hbm.at[pl.ds(off, K)], sbuf[r], sem_i[r])
        pltpu.async_copy(dst_hbm.at[pl.ds(off, K)], dbuf[r], sem_i[r])

    def wait_idx(r):
        pltpu.make_async_copy(src_hbm.at[pl.ds(0, K)], sbuf[r],
                              sem_i[r]).wait()
        pltpu.make_async_copy(dst_hbm.at[pl.ds(0, K)], dbuf[r],
                              sem_i[r]).wait()

    def gather(r):
        pltpu.async_copy(g_hbm.at[sbuf[r]], rows[r], sem_g[r])

    def wait_gather(r):
        pltpu.make_async_copy(g_hbm.at[sbuf[r]], rows[r], sem_g[r]).wait()

    def scatter(r):
        pltpu.sync_copy(rows[r], acc.at[dbuf[r]], add=True)

    with jax.named_scope("agg_edges"):
        for r in range(R):
            load_idx(r, r)
        for r in range(R):
            wait_idx(r)
            gather(r)

        def grp(gi, carry):
            for r in range(R):
                wait_gather(r)
                scatter(r)
                load_idx(R * (gi + 1) + r, r)
            for r in range(R):
                wait_idx(r)
                gather(r)
            return carry

        lax.fori_loop(0, ngw - 1, grp, 0)
        for r in range(R):
            wait_gather(r)
            scatter(r)
        plsc.subcore_barrier()
    with jax.named_scope("agg_out"):
        for j in range(RPT // 128):
            lo = s * RPT + j * 128
            pltpu.sync_copy(acc.at[pl.ds(lo, 128)],
                            out_hbm.at[c, pl.ds(lo, 128)])


@functools.cache
def _sc_aggregate_kernel():
    return pl.kernel(
        _sc_agg_body,
        out_type=jax.ShapeDtypeStruct((NC, NP, H), _f32),
        mesh=plsc.VectorSubcoreMesh(core_axis_name="c", subcore_axis_name="s",
                                    num_cores=NC, num_subcores=NS),
        scratch_types=[
            pltpu.VMEM((K,), jnp.int32),
            pltpu.VMEM((K,), jnp.int32),
            pltpu.VMEM((K,), jnp.int32),
            pltpu.VMEM((K,), jnp.int32),
            pltpu.VMEM((K,), jnp.int32),
            pltpu.VMEM((K,), jnp.int32),
            pltpu.VMEM((K,), jnp.int32),
            pltpu.VMEM((K,), jnp.int32),
            pltpu.VMEM((K, H), _f32),
            pltpu.VMEM((K, H), _f32),
            pltpu.VMEM((K, H), _f32),
            pltpu.VMEM((K, H), _f32),
            pltpu.VMEM_SHARED((NP, H), _f32),
        ] + [pltpu.SemaphoreType.DMA] * (2 * R),
    )


BLK = 1000
GRID = N // BLK
_DOT = (((1,), (1,)), ((), ()))
_DOT0 = (((0,), (0,)), ((), ()))


def _dinv_of(dp_ref):
    deg = dp_ref[0, :, 0:1] + dp_ref[1, :, 0:1] + 1.0
    return lax.rsqrt(deg)


def _prep1_body(x_ref, w_ref, dp_ref, g_ref):
    dinv = _dinv_of(dp_ref)
    h = lax.dot_general(x_ref[...], w_ref[...], _DOT,
                        preferred_element_type=_f32)
    g_ref[...] = h * dinv


def _prep2_body(p_ref, g1_ref, dp_ref, b_ref, w_ref, g2_ref):
    dinv = _dinv_of(dp_ref)
    pre = (p_ref[0] + p_ref[1] + g1_ref[...]) * dinv + b_ref[...]
    h1 = jnp.maximum(pre, 0.0)
    g2_ref[...] = lax.dot_general(h1, w_ref[...], _DOT,
                                  preferred_element_type=_f32) * dinv


def _final_body(q_ref, g2_ref, dp_ref, b_ref, bt_ref, wl_ref, bl_ref,
                o_ref, psum, pcnt):
    i = pl.program_id(0)

    @pl.when(i == 0)
    def _():
        psum[...] = jnp.zeros_like(psum)
        pcnt[...] = jnp.zeros_like(pcnt)

    dinv = _dinv_of(dp_ref)
    pre = (q_ref[0] + q_ref[1] + g2_ref[...]) * dinv + b_ref[...]
    h2 = jnp.maximum(pre, 0.0)
    oh = (lax.broadcasted_iota(jnp.int32, (BLK, G), 1)
          == bt_ref[...]).astype(_f32)
    psum[...] += lax.dot_general(oh, h2, _DOT0, preferred_element_type=_f32)
    pcnt[...] += lax.dot_general(oh, jnp.ones((BLK, H), _f32), _DOT0,
                                 preferred_element_type=_f32)

    @pl.when(i == GRID - 1)
    def _():
        pooled = psum[...] / jnp.maximum(pcnt[...], 1.0)
        o_ref[...] = lax.dot_general(pooled, wl_ref[...], _DOT,
                                     preferred_element_type=_f32) + bl_ref[...]


def _prep1(x, W1, degp):
    return pl.pallas_call(
        _prep1_body,
        grid=(GRID,),
        in_specs=[
            pl.BlockSpec((BLK, D), lambda i: (i, 0)),
            pl.BlockSpec((H, D), lambda i: (0, 0)),
            pl.BlockSpec((NC, BLK, 1), lambda i: (0, i, 0)),
        ],
        out_specs=pl.BlockSpec((BLK, H), lambda i: (i, 0)),
        out_shape=jax.ShapeDtypeStruct((N, H), _f32),
    )(x, W1, degp)


def _prep2(P, g1, degp, b1, W2):
    return pl.pallas_call(
        _prep2_body,
        grid=(GRID,),
        in_specs=[
            pl.BlockSpec((NC, BLK, H), lambda i: (0, i, 0)),
            pl.BlockSpec((BLK, H), lambda i: (i, 0)),
            pl.BlockSpec((NC, BLK, 1), lambda i: (0, i, 0)),
            pl.BlockSpec((1, H), lambda i: (0, 0)),
            pl.BlockSpec((H, H), lambda i: (0, 0)),
        ],
        out_specs=pl.BlockSpec((BLK, H), lambda i: (i, 0)),
        out_shape=jax.ShapeDtypeStruct((N, H), _f32),
    )(P, g1, degp, b1, W2)


def _final(Q, g2, degp, b2, batch2, Wl, bl):
    return pl.pallas_call(
        _final_body,
        grid=(GRID,),
        in_specs=[
            pl.BlockSpec((NC, BLK, H), lambda i: (0, i, 0)),
            pl.BlockSpec((BLK, H), lambda i: (i, 0)),
            pl.BlockSpec((NC, BLK, 1), lambda i: (0, i, 0)),
            pl.BlockSpec((1, H), lambda i: (0, 0)),
            pl.BlockSpec((BLK, 1), lambda i: (i, 0)),
            pl.BlockSpec((O, H), lambda i: (0, 0)),
            pl.BlockSpec((1, O), lambda i: (0, 0)),
        ],
        out_specs=pl.BlockSpec((G, O), lambda i: (0, 0)),
        out_shape=jax.ShapeDtypeStruct((G, O), _f32),
        scratch_shapes=[
            pltpu.VMEM((G, H), _f32),
            pltpu.VMEM((G, H), _f32),
        ],
    )(Q, g2, degp, b2, batch2, Wl, bl)


def _pack_edges(edge_index):
    pad = EP - E
    src_p = jnp.concatenate(
        [edge_index[0], jnp.zeros((pad,), edge_index.dtype)])
    dump = N + jnp.arange(pad, dtype=edge_index.dtype) % (NP - N)
    dst_p = jnp.concatenate([edge_index[1], dump])
    return src_p, dst_p


def kernel(x, edge_index, batch, W1, b1, W2, b2, Wl, bl):
    srcp, dstp = _pack_edges(edge_index)
    degp = _sc_degree_kernel()(dstp).reshape(NC, NP, 1)
    zrs = jnp.zeros((NP, H), _f32)
    g1 = _prep1(x, W1, degp)
    P = _sc_aggregate_kernel()(g1, srcp, dstp, zrs)
    g2 = _prep2(P, g1, degp, b1.reshape(1, H), W2)
    Q = _sc_aggregate_kernel()(g2, srcp, dstp, zrs)
    return _final(Q, g2, degp, b2.reshape(1, H), batch.reshape(N, 1), Wl,
                  bl.reshape(1, O))


# speedup vs baseline: 2.4937x; 2.4937x over previous
import functools

import jax
import jax.numpy as jnp
from jax import lax
from jax.experimental import pallas as pl
from jax.experimental.pallas import tpu as pltpu
from jax.experimental.pallas import tpu_sc as plsc

N = 10000
E = 320000
D = 128
H = 128
O = 128
G = 128

NC, NS, L = 2, 16, 16
NW = NC * NS
K = 80
R = 4
EPWP = 10240
EP = NW * EPWP
NG = EPWP // (R * K)
NP = 10240
EPW0 = 10240
EPW1 = EP // NS - EPW0
NG0 = EPW0 // (R * K)
NG1 = EPW1 // (R * K)
RPT = NP // NS
DW = 16

_f32 = jnp.float32


def _sc_deg_body(dst_hbm, out_hbm, d0, d1, d2, d3, ones_v, zbuf, acc, *sems):
    c = lax.axis_index("c")
    s = lax.axis_index("s")
    w = c * NS + s
    dbuf = (d0, d1, d2, d3)
    sem_i = sems[:R]
    sem_s = sems[R:]

    for i in range(K // L):
        ones_v[pl.ds(i * L, L)] = jnp.ones((L,), _f32)
    for i in range(RPT // L):
        zbuf[pl.ds(i * L, L)] = jnp.zeros((L,), _f32)
    pltpu.sync_copy(zbuf, acc.at[pl.ds(s * RPT, RPT)])
    plsc.subcore_barrier()

    base = w * EPWP

    def load_idx(chunk, r):
        pltpu.async_copy(dst_hbm.at[pl.ds(base + chunk * K, K)], dbuf[r],
                         sem_i[r])

    def wait_idx(r):
        pltpu.make_async_copy(dst_hbm.at[pl.ds(0, K)], dbuf[r],
                              sem_i[r]).wait()

    def scatter(r):
        pltpu.async_copy(ones_v, acc.at[dbuf[r]], sem_s[r], add=True)

    def wait_sc(r):
        pltpu.make_async_copy(ones_v, acc.at[dbuf[r]], sem_s[r]).wait()

    for r in range(R):
        load_idx(r, r)
    for r in range(R):
        wait_idx(r)
        scatter(r)

    def grp(gi, carry):
        for r in range(R):
            wait_sc(r)
            load_idx(R * (gi + 1) + r, r)
        for r in range(R):
            wait_idx(r)
            scatter(r)
        return carry

    lax.fori_loop(0, NG - 1, grp, 0)
    for r in range(R):
        wait_sc(r)
    plsc.subcore_barrier()
    pltpu.sync_copy(acc.at[pl.ds(s * RPT, RPT)],
                    out_hbm.at[c, pl.ds(s * RPT, RPT)])


@functools.cache
def _sc_degree_kernel():
    return pl.kernel(
        _sc_deg_body,
        out_type=jax.ShapeDtypeStruct((NC, NP), _f32),
        mesh=plsc.VectorSubcoreMesh(core_axis_name="c", subcore_axis_name="s",
                                    num_cores=NC, num_subcores=NS),
        scratch_types=[
            pltpu.VMEM((K,), jnp.int32),
            pltpu.VMEM((K,), jnp.int32),
            pltpu.VMEM((K,), jnp.int32),
            pltpu.VMEM((K,), jnp.int32),
            pltpu.VMEM((K,), _f32),
            pltpu.VMEM((RPT,), _f32),
            pltpu.VMEM_SHARED((NP,), _f32),
        ] + [pltpu.SemaphoreType.DMA] * (2 * R),
    )


def _sc_agg_body(g_hbm, src_hbm, dst_hbm, z_hbm, out_hbm,
                 s0, s1, s2, s3, d0, d1, d2, d3,
                 rows0, rows1, rows2, rows3, acc, *sems):
    c = lax.axis_index("c")
    s = lax.axis_index("s")
    w = c * NS + s
    sbuf = (s0, s1, s2, s3)
    dbuf = (d0, d1, d2, d3)
    rows = (rows0, rows1, rows2, rows3)
    sem_i = sems[:R]
    sem_g = sems[R:]

    with jax.named_scope("agg_zero"):
        pltpu.sync_copy(z_hbm.at[pl.ds(s * RPT, RPT)],
                        acc.at[pl.ds(s * RPT, RPT)])
        plsc.subcore_barrier()

    is0 = c == 0
    epw = jnp.where(is0, EPW0, EPW1)
    ngw = jnp.where(is0, NG0, NG1)
    base = c * (NS * EPW0) + s * epw

    def load_idx(chunk, r):
        off = base + chunk * K
        pltpu.async_copy(src_hbm.at[pl.ds(off, K)], sbuf[r], sem_i[r])
        pltpu.async_copy(dst_hbm.at[pl.ds(off, K)], dbuf[r], sem_i[r])

    def wait_idx(r):
        pltpu.make_async_copy(src_hbm.at[pl.ds(0, K)], sbuf[r],
                              sem_i[r]).wait()
        pltpu.make_async_copy(dst_hbm.at[pl.ds(0, K)], dbuf[r],
                              sem_i[r]).wait()

    def gather(r):
        pltpu.async_copy(g_hbm.at[sbuf[r]], rows[r], sem_g[r])

    def wait_gather(r):
        pltpu.make_async_copy(g_hbm.at[sbuf[r]], rows[r], sem_g[r]).wait()

    def scatter(r):
        pltpu.sync_copy(rows[r], acc.at[dbuf[r]], add=True)

    with jax.named_scope("agg_edges"):
        for r in range(R):
            load_idx(r, r)
        for r in range(R):
            wait_idx(r)
            gather(r)

        def grp(gi, carry):
            for r in range(R):
                wait_gather(r)
                scatter(r)
                load_idx(R * (gi + 1) + r, r)
            for r in range(R):
                wait_idx(r)
                gather(r)
            return carry

        lax.fori_loop(0, ngw - 1, grp, 0)
        for r in range(R):
            wait_gather(r)
            scatter(r)
        plsc.subcore_barrier()
    with jax.named_scope("agg_out"):
        for j in range(RPT // 128):
            lo = s * RPT + j * 128
            pltpu.sync_copy(acc.at[pl.ds(lo, 128)],
                            out_hbm.at[c, pl.ds(lo, 128)])


@functools.cache
def _sc_aggregate_kernel():
    return pl.kernel(
        _sc_agg_body,
        out_type=jax.ShapeDtypeStruct((NC, NP, H), _f32),
        mesh=plsc.VectorSubcoreMesh(core_axis_name="c", subcore_axis_name="s",
                                    num_cores=NC, num_subcores=NS),
        scratch_types=[
            pltpu.VMEM((K,), jnp.int32),
            pltpu.VMEM((K,), jnp.int32),
            pltpu.VMEM((K,), jnp.int32),
            pltpu.VMEM((K,), jnp.int32),
            pltpu.VMEM((K,), jnp.int32),
            pltpu.VMEM((K,), jnp.int32),
            pltpu.VMEM((K,), jnp.int32),
            pltpu.VMEM((K,), jnp.int32),
            pltpu.VMEM((K, H), _f32),
            pltpu.VMEM((K, H), _f32),
            pltpu.VMEM((K, H), _f32),
            pltpu.VMEM((K, H), _f32),
            pltpu.VMEM_SHARED((NP, H), _f32),
        ] + [pltpu.SemaphoreType.DMA] * (2 * R),
    )


BLK = 1000
GRID = N // BLK
_DOT = (((1,), (1,)), ((), ()))
_DOT0 = (((0,), (0,)), ((), ()))


def _dinv_of(dp_ref):
    deg = dp_ref[0, :, 0:1] + dp_ref[1, :, 0:1] + 1.0
    return lax.rsqrt(deg)


def _prep1_body(x_ref, w_ref, dp_ref, g_ref):
    dinv = _dinv_of(dp_ref)
    h = lax.dot_general(x_ref[...], w_ref[...], _DOT,
                        preferred_element_type=_f32)
    g_ref[...] = h * dinv


def _prep2_body(p_ref, g1_ref, dp_ref, b_ref, w_ref, g2_ref):
    dinv = _dinv_of(dp_ref)
    pre = (p_ref[0] + p_ref[1] + g1_ref[...]) * dinv + b_ref[...]
    h1 = jnp.maximum(pre, 0.0)
    g2_ref[...] = lax.dot_general(h1, w_ref[...], _DOT,
                                  preferred_element_type=_f32) * dinv


def _final_body(q_ref, g2_ref, dp_ref, b_ref, bt_ref, wl_ref, bl_ref,
                o_ref, psum, pcnt):
    i = pl.program_id(0)

    @pl.when(i == 0)
    def _():
        psum[...] = jnp.zeros_like(psum)
        pcnt[...] = jnp.zeros_like(pcnt)

    dinv = _dinv_of(dp_ref)
    pre = (q_ref[0] + q_ref[1] + g2_ref[...]) * dinv + b_ref[...]
    h2 = jnp.maximum(pre, 0.0)
    oh = (lax.broadcasted_iota(jnp.int32, (BLK, G), 1)
          == bt_ref[...]).astype(_f32)
    psum[...] += lax.dot_general(oh, h2, _DOT0, preferred_element_type=_f32)
    pcnt[...] += lax.dot_general(oh, jnp.ones((BLK, H), _f32), _DOT0,
                                 preferred_element_type=_f32)

    @pl.when(i == GRID - 1)
    def _():
        pooled = psum[...] / jnp.maximum(pcnt[...], 1.0)
        o_ref[...] = lax.dot_general(pooled, wl_ref[...], _DOT,
                                     preferred_element_type=_f32) + bl_ref[...]


def _prep1(x, W1, degp):
    return pl.pallas_call(
        _prep1_body,
        grid=(GRID,),
        in_specs=[
            pl.BlockSpec((BLK, D), lambda i: (i, 0)),
            pl.BlockSpec((H, D), lambda i: (0, 0)),
            pl.BlockSpec((NC, BLK, 1), lambda i: (0, i, 0)),
        ],
        out_specs=pl.BlockSpec((BLK, H), lambda i: (i, 0)),
        out_shape=jax.ShapeDtypeStruct((N, H), _f32),
    )(x, W1, degp)


def _prep2(P, g1, degp, b1, W2):
    return pl.pallas_call(
        _prep2_body,
        grid=(GRID,),
        in_specs=[
            pl.BlockSpec((NC, BLK, H), lambda i: (0, i, 0)),
            pl.BlockSpec((BLK, H), lambda i: (i, 0)),
            pl.BlockSpec((NC, BLK, 1), lambda i: (0, i, 0)),
            pl.BlockSpec((1, H), lambda i: (0, 0)),
            pl.BlockSpec((H, H), lambda i: (0, 0)),
        ],
        out_specs=pl.BlockSpec((BLK, H), lambda i: (i, 0)),
        out_shape=jax.ShapeDtypeStruct((N, H), _f32),
    )(P, g1, degp, b1, W2)


def _final(Q, g2, degp, b2, batch2, Wl, bl):
    return pl.pallas_call(
        _final_body,
        grid=(GRID,),
        in_specs=[
            pl.BlockSpec((NC, BLK, H), lambda i: (0, i, 0)),
            pl.BlockSpec((BLK, H), lambda i: (i, 0)),
            pl.BlockSpec((NC, BLK, 1), lambda i: (0, i, 0)),
            pl.BlockSpec((1, H), lambda i: (0, 0)),
            pl.BlockSpec((BLK, 1), lambda i: (i, 0)),
            pl.BlockSpec((O, H), lambda i: (0, 0)),
            pl.BlockSpec((1, O), lambda i: (0, 0)),
        ],
        out_specs=pl.BlockSpec((G, O), lambda i: (0, 0)),
        out_shape=jax.ShapeDtypeStruct((G, O), _f32),
        scratch_shapes=[
            pltpu.VMEM((G, H), _f32),
            pltpu.VMEM((G, H), _f32),
        ],
    )(Q, g2, degp, b2, batch2, Wl, bl)


def _pack_edges(edge_index):
    pad = EP - E
    spread = jnp.arange(pad, dtype=edge_index.dtype)
    src_p = jnp.concatenate([edge_index[0], spread % N])
    dump = N + spread % (NP - N)
    dst_p = jnp.concatenate([edge_index[1], dump])
    return src_p, dst_p


def kernel(x, edge_index, batch, W1, b1, W2, b2, Wl, bl):
    srcp, dstp = _pack_edges(edge_index)
    degp = _sc_degree_kernel()(dstp).reshape(NC, NP, 1)
    zrs = jnp.zeros((NP, H), _f32)
    g1 = _prep1(x, W1, degp)
    P = _sc_aggregate_kernel()(g1, srcp, dstp, zrs)
    g2 = _prep2(P, g1, degp, b1.reshape(1, H), W2)
    Q = _sc_aggregate_kernel()(g2, srcp, dstp, zrs)
    return _final(Q, g2, degp, b2.reshape(1, H), batch.reshape(N, 1), Wl,
                  bl.reshape(1, O))


# speedup vs baseline: 2.5496x; 1.0224x over previous
import functools

import jax
import jax.numpy as jnp
from jax import lax
from jax.experimental import pallas as pl
from jax.experimental.pallas import tpu as pltpu
from jax.experimental.pallas import tpu_sc as plsc

N = 10000
E = 320000
D = 128
H = 128
O = 128
G = 128

NC, NS, L = 2, 16, 16
NW = NC * NS
K = 80
R = 4
EPWP = 10240
EP = NW * EPWP
NG = EPWP // (R * K)
NP = 10240
EPW0 = 10240
EPW1 = EP // NS - EPW0
NG0 = EPW0 // (R * K)
NG1 = EPW1 // (R * K)
RPT = NP // NS
DW = 16

_f32 = jnp.float32


def _sc_deg_body(dst_hbm, out_hbm, d0, d1, d2, d3, ones_v, zbuf, acc, *sems):
    c = lax.axis_index("c")
    s = lax.axis_index("s")
    w = c * NS + s
    dbuf = (d0, d1, d2, d3)
    sem_i = sems[:R]
    sem_s = sems[R:]

    for i in range(K // L):
        ones_v[pl.ds(i * L, L)] = jnp.ones((L,), _f32)
    for i in range(RPT // L):
        zbuf[pl.ds(i * L, L)] = jnp.zeros((L,), _f32)
    pltpu.sync_copy(zbuf, acc.at[pl.ds(s * RPT, RPT)])
    plsc.subcore_barrier()

    base = w * EPWP

    def load_idx(chunk, r):
        pltpu.async_copy(dst_hbm.at[pl.ds(base + chunk * K, K)], dbuf[r],
                         sem_i[r])

    def wait_idx(r):
        pltpu.make_async_copy(dst_hbm.at[pl.ds(0, K)], dbuf[r],
                              sem_i[r]).wait()

    def scatter(r):
        pltpu.async_copy(ones_v, acc.at[dbuf[r]], sem_s[r], add=True)

    def wait_sc(r):
        pltpu.make_async_copy(ones_v, acc.at[dbuf[r]], sem_s[r]).wait()

    for r in range(R):
        load_idx(r, r)
    for r in range(R):
        wait_idx(r)
        scatter(r)

    def grp(gi, carry):
        for r in range(R):
            wait_sc(r)
            load_idx(R * (gi + 1) + r, r)
        for r in range(R):
            wait_idx(r)
            scatter(r)
        return carry

    lax.fori_loop(0, NG - 1, grp, 0)
    for r in range(R):
        wait_sc(r)
    plsc.subcore_barrier()
    pltpu.sync_copy(acc.at[pl.ds(s * RPT, RPT)],
                    out_hbm.at[c, pl.ds(s * RPT, RPT)])


@functools.cache
def _sc_degree_kernel():
    return pl.kernel(
        _sc_deg_body,
        out_type=jax.ShapeDtypeStruct((NC, NP), _f32),
        mesh=plsc.VectorSubcoreMesh(core_axis_name="c", subcore_axis_name="s",
                                    num_cores=NC, num_subcores=NS),
        scratch_types=[
            pltpu.VMEM((K,), jnp.int32),
            pltpu.VMEM((K,), jnp.int32),
            pltpu.VMEM((K,), jnp.int32),
            pltpu.VMEM((K,), jnp.int32),
            pltpu.VMEM((K,), _f32),
            pltpu.VMEM((RPT,), _f32),
            pltpu.VMEM_SHARED((NP,), _f32),
        ] + [pltpu.SemaphoreType.DMA] * (2 * R),
    )


def _sc_agg_body(g_hbm, src_hbm, dst_hbm, z_hbm, out_hbm,
                 s0, s1, s2, s3, d0, d1, d2, d3,
                 rows0, rows1, rows2, rows3, acc, *sems):
    c = lax.axis_index("c")
    s = lax.axis_index("s")
    w = c * NS + s
    sbuf = (s0, s1, s2, s3)
    dbuf = (d0, d1, d2, d3)
    rows = (rows0, rows1, rows2, rows3)
    sem_i = sems[:R]
    sem_g = sems[R:2 * R]
    sem_s = sems[2 * R:]

    with jax.named_scope("agg_zero"):
        pltpu.sync_copy(z_hbm.at[pl.ds(s * RPT, RPT)],
                        acc.at[pl.ds(s * RPT, RPT)])
        plsc.subcore_barrier()

    is0 = c == 0
    epw = jnp.where(is0, EPW0, EPW1)
    ngw = jnp.where(is0, NG0, NG1)
    base = c * (NS * EPW0) + s * epw

    def load_idx(chunk, r):
        off = base + chunk * K
        pltpu.async_copy(src_hbm.at[pl.ds(off, K)], sbuf[r], sem_i[r])
        pltpu.async_copy(dst_hbm.at[pl.ds(off, K)], dbuf[r], sem_i[r])

    def wait_idx(r):
        pltpu.make_async_copy(src_hbm.at[pl.ds(0, K)], sbuf[r],
                              sem_i[r]).wait()
        pltpu.make_async_copy(dst_hbm.at[pl.ds(0, K)], dbuf[r],
                              sem_i[r]).wait()

    def gather(r):
        pltpu.async_copy(g_hbm.at[sbuf[r]], rows[r], sem_g[r])

    def wait_gather(r):
        pltpu.make_async_copy(g_hbm.at[sbuf[r]], rows[r], sem_g[r]).wait()

    def scatter(r):
        pltpu.async_copy(rows[r], acc.at[dbuf[r]], sem_s[r], add=True)

    def wait_scatter(r):
        pltpu.make_async_copy(rows[r], acc.at[dbuf[r]], sem_s[r]).wait()

    with jax.named_scope("agg_edges"):
        for r in range(R):
            load_idx(r, r)
        for r in range(R):
            wait_idx(r)
            gather(r)

        def grp(gi, carry):
            for r in range(R):
                wait_gather(r)
                scatter(r)
            for r in range(R):
                wait_scatter(r)
                load_idx(R * (gi + 1) + r, r)
            for r in range(R):
                wait_idx(r)
                gather(r)
            return carry

        lax.fori_loop(0, ngw - 1, grp, 0)
        for r in range(R):
            wait_gather(r)
            scatter(r)
        for r in range(R):
            wait_scatter(r)
        plsc.subcore_barrier()
    with jax.named_scope("agg_out"):
        for j in range(RPT // 128):
            lo = s * RPT + j * 128
            pltpu.sync_copy(acc.at[pl.ds(lo, 128)],
                            out_hbm.at[c, pl.ds(lo, 128)])


@functools.cache
def _sc_aggregate_kernel():
    return pl.kernel(
        _sc_agg_body,
        out_type=jax.ShapeDtypeStruct((NC, NP, H), _f32),
        mesh=plsc.VectorSubcoreMesh(core_axis_name="c", subcore_axis_name="s",
                                    num_cores=NC, num_subcores=NS),
        scratch_types=[
            pltpu.VMEM((K,), jnp.int32),
            pltpu.VMEM((K,), jnp.int32),
            pltpu.VMEM((K,), jnp.int32),
            pltpu.VMEM((K,), jnp.int32),
            pltpu.VMEM((K,), jnp.int32),
            pltpu.VMEM((K,), jnp.int32),
            pltpu.VMEM((K,), jnp.int32),
            pltpu.VMEM((K,), jnp.int32),
            pltpu.VMEM((K, H), _f32),
            pltpu.VMEM((K, H), _f32),
            pltpu.VMEM((K, H), _f32),
            pltpu.VMEM((K, H), _f32),
            pltpu.VMEM_SHARED((NP, H), _f32),
        ] + [pltpu.SemaphoreType.DMA] * (3 * R),
    )


BLK = 1000
GRID = N // BLK
_DOT = (((1,), (1,)), ((), ()))
_DOT0 = (((0,), (0,)), ((), ()))


def _dinv_of(dp_ref):
    deg = dp_ref[0, :, 0:1] + dp_ref[1, :, 0:1] + 1.0
    return lax.rsqrt(deg)


def _prep1_body(x_ref, w_ref, dp_ref, g_ref):
    dinv = _dinv_of(dp_ref)
    h = lax.dot_general(x_ref[...], w_ref[...], _DOT,
                        preferred_element_type=_f32)
    g_ref[...] = h * dinv


def _prep2_body(p_ref, g1_ref, dp_ref, b_ref, w_ref, g2_ref):
    dinv = _dinv_of(dp_ref)
    pre = (p_ref[0] + p_ref[1] + g1_ref[...]) * dinv + b_ref[...]
    h1 = jnp.maximum(pre, 0.0)
    g2_ref[...] = lax.dot_general(h1, w_ref[...], _DOT,
                                  preferred_element_type=_f32) * dinv


def _final_body(q_ref, g2_ref, dp_ref, b_ref, bt_ref, wl_ref, bl_ref,
                o_ref, psum, pcnt):
    i = pl.program_id(0)

    @pl.when(i == 0)
    def _():
        psum[...] = jnp.zeros_like(psum)
        pcnt[...] = jnp.zeros_like(pcnt)

    dinv = _dinv_of(dp_ref)
    pre = (q_ref[0] + q_ref[1] + g2_ref[...]) * dinv + b_ref[...]
    h2 = jnp.maximum(pre, 0.0)
    oh = (lax.broadcasted_iota(jnp.int32, (BLK, G), 1)
          == bt_ref[...]).astype(_f32)
    psum[...] += lax.dot_general(oh, h2, _DOT0, preferred_element_type=_f32)
    pcnt[...] += lax.dot_general(oh, jnp.ones((BLK, H), _f32), _DOT0,
                                 preferred_element_type=_f32)

    @pl.when(i == GRID - 1)
    def _():
        pooled = psum[...] / jnp.maximum(pcnt[...], 1.0)
        o_ref[...] = lax.dot_general(pooled, wl_ref[...], _DOT,
                                     preferred_element_type=_f32) + bl_ref[...]


def _prep1(x, W1, degp):
    return pl.pallas_call(
        _prep1_body,
        grid=(GRID,),
        in_specs=[
            pl.BlockSpec((BLK, D), lambda i: (i, 0)),
            pl.BlockSpec((H, D), lambda i: (0, 0)),
            pl.BlockSpec((NC, BLK, 1), lambda i: (0, i, 0)),
        ],
        out_specs=pl.BlockSpec((BLK, H), lambda i: (i, 0)),
        out_shape=jax.ShapeDtypeStruct((N, H), _f32),
    )(x, W1, degp)


def _prep2(P, g1, degp, b1, W2):
    return pl.pallas_call(
        _prep2_body,
        grid=(GRID,),
        in_specs=[
            pl.BlockSpec((NC, BLK, H), lambda i: (0, i, 0)),
            pl.BlockSpec((BLK, H), lambda i: (i, 0)),
            pl.BlockSpec((NC, BLK, 1), lambda i: (0, i, 0)),
            pl.BlockSpec((1, H), lambda i: (0, 0)),
            pl.BlockSpec((H, H), lambda i: (0, 0)),
        ],
        out_specs=pl.BlockSpec((BLK, H), lambda i: (i, 0)),
        out_shape=jax.ShapeDtypeStruct((N, H), _f32),
    )(P, g1, degp, b1, W2)


def _final(Q, g2, degp, b2, batch2, Wl, bl):
    return pl.pallas_call(
        _final_body,
        grid=(GRID,),
        in_specs=[
            pl.BlockSpec((NC, BLK, H), lambda i: (0, i, 0)),
            pl.BlockSpec((BLK, H), lambda i: (i, 0)),
            pl.BlockSpec((NC, BLK, 1), lambda i: (0, i, 0)),
            pl.BlockSpec((1, H), lambda i: (0, 0)),
            pl.BlockSpec((BLK, 1), lambda i: (i, 0)),
            pl.BlockSpec((O, H), lambda i: (0, 0)),
            pl.BlockSpec((1, O), lambda i: (0, 0)),
        ],
        out_specs=pl.BlockSpec((G, O), lambda i: (0, 0)),
        out_shape=jax.ShapeDtypeStruct((G, O), _f32),
        scratch_shapes=[
            pltpu.VMEM((G, H), _f32),
            pltpu.VMEM((G, H), _f32),
        ],
    )(Q, g2, degp, b2, batch2, Wl, bl)


def _pack_edges(edge_index):
    pad = EP - E
    spread = jnp.arange(pad, dtype=edge_index.dtype)
    src_p = jnp.concatenate([edge_index[0], spread % N])
    dump = N + spread % (NP - N)
    dst_p = jnp.concatenate([edge_index[1], dump])
    return src_p, dst_p


def kernel(x, edge_index, batch, W1, b1, W2, b2, Wl, bl):
    srcp, dstp = _pack_edges(edge_index)
    degp = _sc_degree_kernel()(dstp).reshape(NC, NP, 1)
    zrs = jnp.zeros((NP, H), _f32)
    g1 = _prep1(x, W1, degp)
    P = _sc_aggregate_kernel()(g1, srcp, dstp, zrs)
    g2 = _prep2(P, g1, degp, b1.reshape(1, H), W2)
    Q = _sc_aggregate_kernel()(g2, srcp, dstp, zrs)
    return _final(Q, g2, degp, b2.reshape(1, H), batch.reshape(N, 1), Wl,
                  bl.reshape(1, O))


# speedup vs baseline: 2.5508x; 1.0005x over previous
import functools

import jax
import jax.numpy as jnp
from jax import lax
from jax.experimental import pallas as pl
from jax.experimental.pallas import tpu as pltpu
from jax.experimental.pallas import tpu_sc as plsc

N = 10000
E = 320000
D = 128
H = 128
O = 128
G = 128

NC, NS, L = 2, 16, 16
NW = NC * NS
K = 32
R = 8
EPWP = 10240
EP = NW * EPWP
NG = EPWP // (R * K)
NP = 10240
EPW0 = 10240
EPW1 = EP // NS - EPW0
NG0 = EPW0 // (R * K)
NG1 = EPW1 // (R * K)
RPT = NP // NS
DW = 16

_f32 = jnp.float32


def _sc_deg_body(dst_hbm, out_hbm, *rest):
    dbuf = rest[:R]
    ones_v = rest[R]
    zbuf = rest[R + 1]
    acc = rest[R + 2]
    sems = rest[R + 3:]
    c = lax.axis_index("c")
    s = lax.axis_index("s")
    w = c * NS + s
    sem_i = sems[:R]
    sem_s = sems[R:]

    for i in range(K // L):
        ones_v[pl.ds(i * L, L)] = jnp.ones((L,), _f32)
    for i in range(RPT // L):
        zbuf[pl.ds(i * L, L)] = jnp.zeros((L,), _f32)
    pltpu.sync_copy(zbuf, acc.at[pl.ds(s * RPT, RPT)])
    plsc.subcore_barrier()

    base = w * EPWP

    def load_idx(chunk, r):
        pltpu.async_copy(dst_hbm.at[pl.ds(base + chunk * K, K)], dbuf[r],
                         sem_i[r])

    def wait_idx(r):
        pltpu.make_async_copy(dst_hbm.at[pl.ds(0, K)], dbuf[r],
                              sem_i[r]).wait()

    def scatter(r):
        pltpu.async_copy(ones_v, acc.at[dbuf[r]], sem_s[r], add=True)

    def wait_sc(r):
        pltpu.make_async_copy(ones_v, acc.at[dbuf[r]], sem_s[r]).wait()

    for r in range(R):
        load_idx(r, r)
    for r in range(R):
        wait_idx(r)
        scatter(r)

    def grp(gi, carry):
        for r in range(R):
            wait_sc(r)
            load_idx(R * (gi + 1) + r, r)
        for r in range(R):
            wait_idx(r)
            scatter(r)
        return carry

    lax.fori_loop(0, NG - 1, grp, 0)
    for r in range(R):
        wait_sc(r)
    plsc.subcore_barrier()
    pltpu.sync_copy(acc.at[pl.ds(s * RPT, RPT)],
                    out_hbm.at[c, pl.ds(s * RPT, RPT)])


@functools.cache
def _sc_degree_kernel():
    return pl.kernel(
        _sc_deg_body,
        out_type=jax.ShapeDtypeStruct((NC, NP), _f32),
        mesh=plsc.VectorSubcoreMesh(core_axis_name="c", subcore_axis_name="s",
                                    num_cores=NC, num_subcores=NS),
        scratch_types=(
            [pltpu.VMEM((K,), jnp.int32)] * R
            + [pltpu.VMEM((K,), _f32),
               pltpu.VMEM((RPT,), _f32),
               pltpu.VMEM_SHARED((NP,), _f32)]
            + [pltpu.SemaphoreType.DMA] * (2 * R)
        ),
    )


def _sc_agg_body(g_hbm, src_hbm, dst_hbm, z_hbm, out_hbm, *rest):
    sbuf = rest[:R]
    dbuf = rest[R:2 * R]
    rows = rest[2 * R:3 * R]
    acc = rest[3 * R]
    sems = rest[3 * R + 1:]
    c = lax.axis_index("c")
    s = lax.axis_index("s")
    w = c * NS + s
    sem_i = sems[:R]
    sem_g = sems[R:2 * R]
    sem_s = sems[2 * R:]

    with jax.named_scope("agg_zero"):
        pltpu.sync_copy(z_hbm.at[pl.ds(s * RPT, RPT)],
                        acc.at[pl.ds(s * RPT, RPT)])
        plsc.subcore_barrier()

    is0 = c == 0
    epw = jnp.where(is0, EPW0, EPW1)
    ngw = jnp.where(is0, NG0, NG1)
    base = c * (NS * EPW0) + s * epw

    def load_idx(chunk, r):
        off = base + chunk * K
        pltpu.async_copy(src_hbm.at[pl.ds(off, K)], sbuf[r], sem_i[r])
        pltpu.async_copy(dst_hbm.at[pl.ds(off, K)], dbuf[r], sem_i[r])

    def wait_idx(r):
        pltpu.make_async_copy(src_hbm.at[pl.ds(0, K)], sbuf[r],
                              sem_i[r]).wait()
        pltpu.make_async_copy(dst_hbm.at[pl.ds(0, K)], dbuf[r],
                              sem_i[r]).wait()

    def gather(r):
        pltpu.async_copy(g_hbm.at[sbuf[r]], rows[r], sem_g[r])

    def wait_gather(r):
        pltpu.make_async_copy(g_hbm.at[sbuf[r]], rows[r], sem_g[r]).wait()

    def scatter(r):
        pltpu.async_copy(rows[r], acc.at[dbuf[r]], sem_s[r], add=True)

    def wait_scatter(r):
        pltpu.make_async_copy(rows[r], acc.at[dbuf[r]], sem_s[r]).wait()

    with jax.named_scope("agg_edges"):
        for r in range(R):
            load_idx(r, r)
        for r in range(R):
            wait_idx(r)
            gather(r)

        def grp(gi, carry):
            for r in range(R):
                wait_gather(r)
                scatter(r)
            for r in range(R):
                wait_scatter(r)
                load_idx(R * (gi + 1) + r, r)
            for r in range(R):
                wait_idx(r)
                gather(r)
            return carry

        lax.fori_loop(0, ngw - 1, grp, 0)
        for r in range(R):
            wait_gather(r)
            scatter(r)
        for r in range(R):
            wait_scatter(r)
        plsc.subcore_barrier()
    with jax.named_scope("agg_out"):
        for j in range(RPT // 128):
            lo = s * RPT + j * 128
            pltpu.sync_copy(acc.at[pl.ds(lo, 128)],
                            out_hbm.at[c, pl.ds(lo, 128)])


@functools.cache
def _sc_aggregate_kernel():
    return pl.kernel(
        _sc_agg_body,
        out_type=jax.ShapeDtypeStruct((NC, NP, H), _f32),
        mesh=plsc.VectorSubcoreMesh(core_axis_name="c", subcore_axis_name="s",
                                    num_cores=NC, num_subcores=NS),
        scratch_types=(
            [pltpu.VMEM((K,), jnp.int32)] * (2 * R)
            + [pltpu.VMEM((K, H), _f32)] * R
            + [pltpu.VMEM_SHARED((NP, H), _f32)]
            + [pltpu.SemaphoreType.DMA] * (3 * R)
        ),
    )


BLK = 1000
GRID = N // BLK
_DOT = (((1,), (1,)), ((), ()))
_DOT0 = (((0,), (0,)), ((), ()))


def _dinv_of(dp_ref):
    deg = dp_ref[0, :, 0:1] + dp_ref[1, :, 0:1] + 1.0
    return lax.rsqrt(deg)


def _prep1_body(x_ref, w_ref, dp_ref, g_ref):
    dinv = _dinv_of(dp_ref)
    h = lax.dot_general(x_ref[...], w_ref[...], _DOT,
                        preferred_element_type=_f32)
    g_ref[...] = h * dinv


def _prep2_body(p_ref, g1_ref, dp_ref, b_ref, w_ref, g2_ref):
    dinv = _dinv_of(dp_ref)
    pre = (p_ref[0] + p_ref[1] + g1_ref[...]) * dinv + b_ref[...]
    h1 = jnp.maximum(pre, 0.0)
    g2_ref[...] = lax.dot_general(h1, w_ref[...], _DOT,
                                  preferred_element_type=_f32) * dinv


def _final_body(q_ref, g2_ref, dp_ref, b_ref, bt_ref, wl_ref, bl_ref,
                o_ref, psum, pcnt):
    i = pl.program_id(0)

    @pl.when(i == 0)
    def _():
        psum[...] = jnp.zeros_like(psum)
        pcnt[...] = jnp.zeros_like(pcnt)

    dinv = _dinv_of(dp_ref)
    pre = (q_ref[0] + q_ref[1] + g2_ref[...]) * dinv + b_ref[...]
    h2 = jnp.maximum(pre, 0.0)
    oh = (lax.broadcasted_iota(jnp.int32, (BLK, G), 1)
          == bt_ref[...]).astype(_f32)
    psum[...] += lax.dot_general(oh, h2, _DOT0, preferred_element_type=_f32)
    pcnt[...] += lax.dot_general(oh, jnp.ones((BLK, H), _f32), _DOT0,
                                 preferred_element_type=_f32)

    @pl.when(i == GRID - 1)
    def _():
        pooled = psum[...] / jnp.maximum(pcnt[...], 1.0)
        o_ref[...] = lax.dot_general(pooled, wl_ref[...], _DOT,
                                     preferred_element_type=_f32) + bl_ref[...]


def _prep1(x, W1, degp):
    return pl.pallas_call(
        _prep1_body,
        grid=(GRID,),
        in_specs=[
            pl.BlockSpec((BLK, D), lambda i: (i, 0)),
            pl.BlockSpec((H, D), lambda i: (0, 0)),
            pl.BlockSpec((NC, BLK, 1), lambda i: (0, i, 0)),
        ],
        out_specs=pl.BlockSpec((BLK, H), lambda i: (i, 0)),
        out_shape=jax.ShapeDtypeStruct((N, H), _f32),
    )(x, W1, degp)


def _prep2(P, g1, degp, b1, W2):
    return pl.pallas_call(
        _prep2_body,
        grid=(GRID,),
        in_specs=[
            pl.BlockSpec((NC, BLK, H), lambda i: (0, i, 0)),
            pl.BlockSpec((BLK, H), lambda i: (i, 0)),
            pl.BlockSpec((NC, BLK, 1), lambda i: (0, i, 0)),
            pl.BlockSpec((1, H), lambda i: (0, 0)),
            pl.BlockSpec((H, H), lambda i: (0, 0)),
        ],
        out_specs=pl.BlockSpec((BLK, H), lambda i: (i, 0)),
        out_shape=jax.ShapeDtypeStruct((N, H), _f32),
    )(P, g1, degp, b1, W2)


def _final(Q, g2, degp, b2, batch2, Wl, bl):
    return pl.pallas_call(
        _final_body,
        grid=(GRID,),
        in_specs=[
            pl.BlockSpec((NC, BLK, H), lambda i: (0, i, 0)),
            pl.BlockSpec((BLK, H), lambda i: (i, 0)),
            pl.BlockSpec((NC, BLK, 1), lambda i: (0, i, 0)),
            pl.BlockSpec((1, H), lambda i: (0, 0)),
            pl.BlockSpec((BLK, 1), lambda i: (i, 0)),
            pl.BlockSpec((O, H), lambda i: (0, 0)),
            pl.BlockSpec((1, O), lambda i: (0, 0)),
        ],
        out_specs=pl.BlockSpec((G, O), lambda i: (0, 0)),
        out_shape=jax.ShapeDtypeStruct((G, O), _f32),
        scratch_shapes=[
            pltpu.VMEM((G, H), _f32),
            pltpu.VMEM((G, H), _f32),
        ],
    )(Q, g2, degp, b2, batch2, Wl, bl)


def _pack_edges(edge_index):
    pad = EP - E
    spread = jnp.arange(pad, dtype=edge_index.dtype)
    src_p = jnp.concatenate([edge_index[0], spread % N])
    dump = N + spread % (NP - N)
    dst_p = jnp.concatenate([edge_index[1], dump])
    return src_p, dst_p


def kernel(x, edge_index, batch, W1, b1, W2, b2, Wl, bl):
    srcp, dstp = _pack_edges(edge_index)
    degp = _sc_degree_kernel()(dstp).reshape(NC, NP, 1)
    zrs = jnp.zeros((NP, H), _f32)
    g1 = _prep1(x, W1, degp)
    P = _sc_aggregate_kernel()(g1, srcp, dstp, zrs)
    g2 = _prep2(P, g1, degp, b1.reshape(1, H), W2)
    Q = _sc_aggregate_kernel()(g2, srcp, dstp, zrs)
    return _final(Q, g2, degp, b2.reshape(1, H), batch.reshape(N, 1), Wl,
                  bl.reshape(1, O))


# speedup vs baseline: 2.5864x; 1.0140x over previous
import functools

import jax
import jax.numpy as jnp
from jax import lax
from jax.experimental import pallas as pl
from jax.experimental.pallas import tpu as pltpu
from jax.experimental.pallas import tpu_sc as plsc

N = 10000
E = 320000
D = 128
H = 128
O = 128
G = 128

NC, NS, L = 2, 16, 16
NW = NC * NS
K = 32
R = 8
EPWP = 10240
EP = NW * EPWP
NG = EPWP // (R * K)
NP = 10240
EPW0 = 10240
EPW1 = EP // NS - EPW0
NG0 = EPW0 // (R * K)
NG1 = EPW1 // (R * K)
RPT = NP // NS
DW = 16

_f32 = jnp.float32


def _sc_deg_body(dst_hbm, out_hbm, *rest):
    dbuf = rest[:R]
    ones_v = rest[R]
    zbuf = rest[R + 1]
    acc = rest[R + 2]
    sems = rest[R + 3:]
    c = lax.axis_index("c")
    s = lax.axis_index("s")
    w = c * NS + s
    sem_i = sems[:R]
    sem_s = sems[R:]

    for i in range(K // L):
        ones_v[pl.ds(i * L, L)] = jnp.ones((L,), _f32)
    for i in range(RPT // L):
        zbuf[pl.ds(i * L, L)] = jnp.zeros((L,), _f32)
    pltpu.sync_copy(zbuf, acc.at[pl.ds(s * RPT, RPT)])
    plsc.subcore_barrier()

    base = w * EPWP

    def load_idx(chunk, r):
        pltpu.async_copy(dst_hbm.at[pl.ds(base + chunk * K, K)], dbuf[r],
                         sem_i[r])

    def wait_idx(r):
        pltpu.make_async_copy(dst_hbm.at[pl.ds(0, K)], dbuf[r],
                              sem_i[r]).wait()

    def scatter(r):
        pltpu.async_copy(ones_v, acc.at[dbuf[r]], sem_s[r], add=True)

    def wait_sc(r):
        pltpu.make_async_copy(ones_v, acc.at[dbuf[r]], sem_s[r]).wait()

    for r in range(R):
        load_idx(r, r)
    for r in range(R):
        wait_idx(r)
        scatter(r)

    def grp(gi, carry):
        for r in range(R):
            wait_sc(r)
            load_idx(R * (gi + 1) + r, r)
        for r in range(R):
            wait_idx(r)
            scatter(r)
        return carry

    lax.fori_loop(0, NG - 1, grp, 0)
    for r in range(R):
        wait_sc(r)
    plsc.subcore_barrier()
    pltpu.sync_copy(acc.at[pl.ds(s * RPT, RPT)],
                    out_hbm.at[c, pl.ds(s * RPT, RPT)])


@functools.cache
def _sc_degree_kernel():
    return pl.kernel(
        _sc_deg_body,
        out_type=jax.ShapeDtypeStruct((NC, NP), _f32),
        mesh=plsc.VectorSubcoreMesh(core_axis_name="c", subcore_axis_name="s",
                                    num_cores=NC, num_subcores=NS),
        scratch_types=(
            [pltpu.VMEM((K,), jnp.int32)] * R
            + [pltpu.VMEM((K,), _f32),
               pltpu.VMEM((RPT,), _f32),
               pltpu.VMEM_SHARED((NP,), _f32)]
            + [pltpu.SemaphoreType.DMA] * (2 * R)
        ),
    )


def _sc_agg_body(g_hbm, src_hbm, dst_hbm, z_hbm, out_hbm, *rest):
    sbuf = rest[:R]
    dbuf = rest[R:2 * R]
    rows = rest[2 * R:3 * R]
    acc = rest[3 * R]
    sems = rest[3 * R + 1:]
    c = lax.axis_index("c")
    s = lax.axis_index("s")
    w = c * NS + s
    sem_i = sems[:R]
    sem_g = sems[R:2 * R]
    sem_s = sems[2 * R:]

    with jax.named_scope("agg_zero"):
        pltpu.sync_copy(z_hbm.at[pl.ds(s * RPT, RPT)],
                        acc.at[pl.ds(s * RPT, RPT)])
        plsc.subcore_barrier()

    is0 = c == 0
    epw = jnp.where(is0, EPW0, EPW1)
    ngw = jnp.where(is0, NG0, NG1)
    base = c * (NS * EPW0) + s * epw

    def load_idx(chunk, r):
        off = base + chunk * K
        pltpu.async_copy(src_hbm.at[pl.ds(off, K)], sbuf[r], sem_i[r])
        pltpu.async_copy(dst_hbm.at[pl.ds(off, K)], dbuf[r], sem_i[r])

    def wait_idx(r):
        pltpu.make_async_copy(src_hbm.at[pl.ds(0, K)], sbuf[r],
                              sem_i[r]).wait()
        pltpu.make_async_copy(dst_hbm.at[pl.ds(0, K)], dbuf[r],
                              sem_i[r]).wait()

    def gather(r):
        pltpu.async_copy(g_hbm.at[sbuf[r]], rows[r], sem_g[r])

    def wait_gather(r):
        pltpu.make_async_copy(g_hbm.at[sbuf[r]], rows[r], sem_g[r]).wait()

    def scatter(r):
        pltpu.async_copy(rows[r], acc.at[dbuf[r]], sem_s[r], add=True)

    def wait_scatter(r):
        pltpu.make_async_copy(rows[r], acc.at[dbuf[r]], sem_s[r]).wait()

    with jax.named_scope("agg_edges"):
        for r in range(R):
            load_idx(r, r)
        for r in range(R):
            wait_idx(r)
            gather(r)

        def grp(gi, carry):
            for r in range(R):
                wait_gather(r)
                scatter(r)
            for r in range(R):
                wait_scatter(r)
                load_idx(R * (gi + 1) + r, r)
            for r in range(R):
                wait_idx(r)
                gather(r)
            return carry

        lax.fori_loop(0, ngw - 1, grp, 0)
        for r in range(R):
            wait_gather(r)
            scatter(r)
        for r in range(R):
            wait_scatter(r)
        plsc.subcore_barrier()
    with jax.named_scope("agg_out"):
        for j in range(RPT // 128):
            lo = s * RPT + j * 128
            pltpu.sync_copy(acc.at[pl.ds(lo, 128)],
                            out_hbm.at[c, pl.ds(lo, 128)])


@functools.cache
def _sc_aggregate_kernel():
    return pl.kernel(
        _sc_agg_body,
        out_type=jax.ShapeDtypeStruct((NC, NP, H), _f32),
        mesh=plsc.VectorSubcoreMesh(core_axis_name="c", subcore_axis_name="s",
                                    num_cores=NC, num_subcores=NS),
        scratch_types=(
            [pltpu.VMEM((K,), jnp.int32)] * (2 * R)
            + [pltpu.VMEM((K, H), _f32)] * R
            + [pltpu.VMEM_SHARED((NP, H), _f32)]
            + [pltpu.SemaphoreType.DMA] * (3 * R)
        ),
    )


BLK = 2000
GRID = N // BLK
_DOT = (((1,), (1,)), ((), ()))
_DOT0 = (((0,), (0,)), ((), ()))


def _dinv_of(dp_ref):
    deg = dp_ref[0, :, 0:1] + dp_ref[1, :, 0:1] + 1.0
    return lax.rsqrt(deg)


def _mm1_body(x_ref, w_ref, h_ref):
    h_ref[...] = lax.dot_general(x_ref[...], w_ref[...], _DOT,
                                 preferred_element_type=_f32)


def _scale_body(h_ref, dp_ref, g_ref):
    g_ref[...] = h_ref[...] * _dinv_of(dp_ref)


def _prep2_body(p_ref, g1_ref, dp_ref, b_ref, w_ref, g2_ref):
    dinv = _dinv_of(dp_ref)
    pre = (p_ref[0] + p_ref[1] + g1_ref[...]) * dinv + b_ref[...]
    h1 = jnp.maximum(pre, 0.0)
    g2_ref[...] = lax.dot_general(h1, w_ref[...], _DOT,
                                  preferred_element_type=_f32) * dinv


def _final_body(q_ref, g2_ref, dp_ref, b_ref, bt_ref, wl_ref, bl_ref,
                o_ref, psum, pcnt):
    i = pl.program_id(0)

    @pl.when(i == 0)
    def _():
        psum[...] = jnp.zeros_like(psum)
        pcnt[...] = jnp.zeros_like(pcnt)

    dinv = _dinv_of(dp_ref)
    pre = (q_ref[0] + q_ref[1] + g2_ref[...]) * dinv + b_ref[...]
    h2 = jnp.maximum(pre, 0.0)
    oh = (lax.broadcasted_iota(jnp.int32, (BLK, G), 1)
          == bt_ref[...]).astype(_f32)
    psum[...] += lax.dot_general(oh, h2, _DOT0, preferred_element_type=_f32)
    pcnt[...] += lax.dot_general(oh, jnp.ones((BLK, H), _f32), _DOT0,
                                 preferred_element_type=_f32)

    @pl.when(i == GRID - 1)
    def _():
        pooled = psum[...] / jnp.maximum(pcnt[...], 1.0)
        o_ref[...] = lax.dot_general(pooled, wl_ref[...], _DOT,
                                     preferred_element_type=_f32) + bl_ref[...]


def _mm1(x, W1):
    return pl.pallas_call(
        _mm1_body,
        grid=(GRID,),
        in_specs=[
            pl.BlockSpec((BLK, D), lambda i: (i, 0)),
            pl.BlockSpec((H, D), lambda i: (0, 0)),
        ],
        out_specs=pl.BlockSpec((BLK, H), lambda i: (i, 0)),
        out_shape=jax.ShapeDtypeStruct((N, H), _f32),
    )(x, W1)


def _scale(h, degp):
    return pl.pallas_call(
        _scale_body,
        grid=(GRID,),
        in_specs=[
            pl.BlockSpec((BLK, H), lambda i: (i, 0)),
            pl.BlockSpec((NC, BLK, 1), lambda i: (0, i, 0)),
        ],
        out_specs=pl.BlockSpec((BLK, H), lambda i: (i, 0)),
        out_shape=jax.ShapeDtypeStruct((N, H), _f32),
    )(h, degp)


def _prep2(P, g1, degp, b1, W2):
    return pl.pallas_call(
        _prep2_body,
        grid=(GRID,),
        in_specs=[
            pl.BlockSpec((NC, BLK, H), lambda i: (0, i, 0)),
            pl.BlockSpec((BLK, H), lambda i: (i, 0)),
            pl.BlockSpec((NC, BLK, 1), lambda i: (0, i, 0)),
            pl.BlockSpec((1, H), lambda i: (0, 0)),
            pl.BlockSpec((H, H), lambda i: (0, 0)),
        ],
        out_specs=pl.BlockSpec((BLK, H), lambda i: (i, 0)),
        out_shape=jax.ShapeDtypeStruct((N, H), _f32),
    )(P, g1, degp, b1, W2)


def _final(Q, g2, degp, b2, batch2, Wl, bl):
    return pl.pallas_call(
        _final_body,
        grid=(GRID,),
        in_specs=[
            pl.BlockSpec((NC, BLK, H), lambda i: (0, i, 0)),
            pl.BlockSpec((BLK, H), lambda i: (i, 0)),
            pl.BlockSpec((NC, BLK, 1), lambda i: (0, i, 0)),
            pl.BlockSpec((1, H), lambda i: (0, 0)),
            pl.BlockSpec((BLK, 1), lambda i: (i, 0)),
            pl.BlockSpec((O, H), lambda i: (0, 0)),
            pl.BlockSpec((1, O), lambda i: (0, 0)),
        ],
        out_specs=pl.BlockSpec((G, O), lambda i: (0, 0)),
        out_shape=jax.ShapeDtypeStruct((G, O), _f32),
        scratch_shapes=[
            pltpu.VMEM((G, H), _f32),
            pltpu.VMEM((G, H), _f32),
        ],
    )(Q, g2, degp, b2, batch2, Wl, bl)


def _pack_edges(edge_index):
    pad = EP - E
    spread = jnp.arange(pad, dtype=edge_index.dtype)
    src_p = jnp.concatenate([edge_index[0], spread % N])
    dump = N + spread % (NP - N)
    dst_p = jnp.concatenate([edge_index[1], dump])
    return src_p, dst_p


def kernel(x, edge_index, batch, W1, b1, W2, b2, Wl, bl):
    srcp, dstp = _pack_edges(edge_index)
    h1raw = _mm1(x, W1)
    degp = _sc_degree_kernel()(dstp).reshape(NC, NP, 1)
    zrs = jnp.zeros((NP, H), _f32)
    g1 = _scale(h1raw, degp)
    P = _sc_aggregate_kernel()(g1, srcp, dstp, zrs)
    g2 = _prep2(P, g1, degp, b1.reshape(1, H), W2)
    Q = _sc_aggregate_kernel()(g2, srcp, dstp, zrs)
    return _final(Q, g2, degp, b2.reshape(1, H), batch.reshape(N, 1), Wl,
                  bl.reshape(1, O))
